# baseline (speedup 1.0000x reference)
import functools

import jax
import jax.numpy as jnp
from jax import lax
from jax.experimental import pallas as pl
from jax.experimental.pallas import tpu as pltpu
from jax.experimental.pallas import tpu_sc as plsc

DIM_HALF = 64
DC = 2 * DIM_HALF
LBASE = 128
SEQ = 8192

NC = 2
NS = 16
NW = NC * NS
BPW = SEQ // NW
VL = 16

SEQ_BLK = 1024
N_SEQ_BLKS = SEQ // SEQ_BLK


def _tables_body(invf_ref, tab_ref):
    l = (lax.broadcasted_iota(jnp.int32, (LBASE, DIM_HALF), 0)
         .astype(jnp.float32))
    ang_lo = l * invf_ref[...]
    ang_hi = ang_lo * float(LBASE)
    tab_ref[0:LBASE, :] = jnp.concatenate(
        [jnp.cos(ang_lo), jnp.sin(ang_lo)], axis=1)
    tab_ref[LBASE:2 * LBASE, :] = jnp.concatenate(
        [jnp.cos(ang_hi), jnp.sin(ang_hi)], axis=1)


def _build_tables(inv_freq):
    invf2d = inv_freq.reshape(1, DIM_HALF)
    return pl.pallas_call(
        _tables_body,
        out_shape=jax.ShapeDtypeStruct((2 * LBASE, DC), jnp.float32),
    )(invf2d)


@functools.cache
def _make_sc_gather():
    mesh = plsc.VectorSubcoreMesh(core_axis_name="c", subcore_axis_name="s")

    @functools.partial(
        pl.kernel,
        mesh=mesh,
        out_type=(
            jax.ShapeDtypeStruct((SEQ, DC), jnp.float32),
            jax.ShapeDtypeStruct((SEQ, DC), jnp.float32),
        ),
        scratch_types=[
            pltpu.VMEM((BPW,), jnp.int32),
            pltpu.VMEM((2 * BPW,), jnp.int32),
            pltpu.VMEM((2 * BPW, DC), jnp.float32),
            pltpu.SemaphoreType.DMA,
        ],
    )
    def _sc_gather(tab_hbm, pos_hbm, lo_out, hi_out,
                   idx_v, idx2_v, rows_v, sem):
        wid = lax.axis_index("s") * NC + lax.axis_index("c")
        base = wid * BPW
        pltpu.sync_copy(pos_hbm.at[pl.ds(base, BPW)], idx_v)
        for k in range(BPW // VL):
            p = idx_v[pl.ds(k * VL, VL)]
            idx2_v[pl.ds(k * VL, VL)] = lax.bitwise_and(p, LBASE - 1)
            idx2_v[pl.ds(BPW + k * VL, VL)] = (
                lax.shift_right_logical(p, 7) + LBASE)
        pltpu.async_copy(tab_hbm.at[idx2_v], rows_v, sem).wait()
        pltpu.sync_copy(rows_v.at[pl.ds(0, BPW)],
                        lo_out.at[pl.ds(base, BPW)])
        pltpu.sync_copy(rows_v.at[pl.ds(BPW, BPW)],
                        hi_out.at[pl.ds(base, BPW)])

    return _sc_gather


def _combine_body(lo_ref, hi_ref, cos_ref, sin_ref):
    cl = lo_ref[:, :DIM_HALF]
    sl = lo_ref[:, DIM_HALF:]
    ch = hi_ref[:, :DIM_HALF]
    sh = hi_ref[:, DIM_HALF:]
    cos_ref[...] = ch * cl - sh * sl
    sin_ref[...] = sh * cl + ch * sl


def _combine(lo_rows, hi_rows):
    return pl.pallas_call(
        _combine_body,
        grid=(N_SEQ_BLKS,),
        in_specs=[
            pl.BlockSpec((SEQ_BLK, DC), lambda i: (i, 0)),
            pl.BlockSpec((SEQ_BLK, DC), lambda i: (i, 0)),
        ],
        out_specs=[
            pl.BlockSpec((SEQ_BLK, DIM_HALF), lambda i: (i, 0)),
            pl.BlockSpec((SEQ_BLK, DIM_HALF), lambda i: (i, 0)),
        ],
        out_shape=[
            jax.ShapeDtypeStruct((SEQ, DIM_HALF), jnp.float32),
            jax.ShapeDtypeStruct((SEQ, DIM_HALF), jnp.float32),
        ],
    )(lo_rows, hi_rows)


def kernel(positions, inv_freq):
    tab = _build_tables(inv_freq)
    pos32 = positions.astype(jnp.int32)
    lo_rows, hi_rows = _make_sc_gather()(tab, pos32)
    cos, sin = _combine(lo_rows, hi_rows)
    return (cos, sin)


# speedup vs baseline: 1.4376x; 1.4376x over previous
import functools

import jax
import jax.numpy as jnp
from jax import lax
from jax.experimental import pallas as pl
from jax.experimental.pallas import tpu as pltpu
from jax.experimental.pallas import tpu_sc as plsc

DIM_HALF = 64
DC = 2 * DIM_HALF
LBASE = 128
EXT = 9216
SEQ = 8192

NC = 2
NS = 16
NW = NC * NS
BPW = SEQ // NW

HPB = 8
CACHE_BLK = HPB * LBASE
N_CACHE_BLKS = EXT // CACHE_BLK

SEQ_BLK = 1024
N_SEQ_BLKS = SEQ // SEQ_BLK


def _tables_body(invf_ref, tab_ref):
    l = (lax.broadcasted_iota(jnp.int32, (LBASE, DIM_HALF), 0)
         .astype(jnp.float32))
    ang_lo = l * invf_ref[...]
    ang_hi = ang_lo * float(LBASE)
    tab_ref[0:LBASE, :] = jnp.concatenate(
        [jnp.cos(ang_lo), jnp.sin(ang_lo)], axis=1)
    tab_ref[LBASE:2 * LBASE, :] = jnp.concatenate(
        [jnp.cos(ang_hi), jnp.sin(ang_hi)], axis=1)


def _combine_body(tab_ref, out_ref):
    i = pl.program_id(0)
    hi = tab_ref[pl.ds(LBASE + HPB * i, HPB), :]
    ch = hi[:, :DIM_HALF].reshape(HPB, 1, DIM_HALF)
    sh = hi[:, DIM_HALF:].reshape(HPB, 1, DIM_HALF)
    lo = tab_ref[0:LBASE, :]
    cl = lo[:, :DIM_HALF].reshape(1, LBASE, DIM_HALF)
    sl = lo[:, DIM_HALF:].reshape(1, LBASE, DIM_HALF)
    cos_c = ch * cl - sh * sl
    sin_c = sh * cl + ch * sl
    out = jnp.concatenate([cos_c, sin_c], axis=2)
    out_ref[...] = out.reshape(CACHE_BLK, DC)


def _build_cache(inv_freq):
    invf2d = inv_freq.reshape(1, DIM_HALF)
    tab = pl.pallas_call(
        _tables_body,
        out_shape=jax.ShapeDtypeStruct((2 * LBASE, DC), jnp.float32),
    )(invf2d)
    return pl.pallas_call(
        _combine_body,
        grid=(N_CACHE_BLKS,),
        in_specs=[pl.BlockSpec((2 * LBASE, DC), lambda i: (0, 0))],
        out_specs=pl.BlockSpec((CACHE_BLK, DC), lambda i: (i, 0)),
        out_shape=jax.ShapeDtypeStruct((EXT, DC), jnp.float32),
    )(tab)


@functools.cache
def _make_sc_gather():
    mesh = plsc.VectorSubcoreMesh(core_axis_name="c", subcore_axis_name="s")

    @functools.partial(
        pl.kernel,
        mesh=mesh,
        out_type=jax.ShapeDtypeStruct((SEQ, DC), jnp.float32),
        scratch_types=[
            pltpu.VMEM((BPW,), jnp.int32),
            pltpu.VMEM((BPW, DC), jnp.float32),
            pltpu.SemaphoreType.DMA,
        ],
    )
    def _sc_gather(cache_hbm, pos_hbm, out_hbm, idx_v, rows_v, sem):
        wid = lax.axis_index("s") * NC + lax.axis_index("c")
        base = wid * BPW
        pltpu.sync_copy(pos_hbm.at[pl.ds(base, BPW)], idx_v)
        pltpu.async_copy(cache_hbm.at[idx_v], rows_v, sem).wait()
        pltpu.sync_copy(rows_v, out_hbm.at[pl.ds(base, BPW)])

    return _sc_gather


def _split_body(both_ref, cos_ref, sin_ref):
    b = both_ref[...]
    cos_ref[...] = b[:, :DIM_HALF].T
    sin_ref[...] = b[:, DIM_HALF:].T


def _split_transpose(both):
    return pl.pallas_call(
        _split_body,
        grid=(N_SEQ_BLKS,),
        in_specs=[pl.BlockSpec((SEQ_BLK, DC), lambda i: (i, 0))],
        out_specs=[
            pl.BlockSpec((DIM_HALF, SEQ_BLK), lambda i: (0, i)),
            pl.BlockSpec((DIM_HALF, SEQ_BLK), lambda i: (0, i)),
        ],
        out_shape=[
            jax.ShapeDtypeStruct((DIM_HALF, SEQ), jnp.float32),
            jax.ShapeDtypeStruct((DIM_HALF, SEQ), jnp.float32),
        ],
    )(both)


def kernel(positions, inv_freq):
    cache = _build_cache(inv_freq)
    pos32 = positions.astype(jnp.int32)
    both = _make_sc_gather()(cache, pos32)
    cos_t, sin_t = _split_transpose(both)
    return (cos_t.T, sin_t.T)
